# baseline (speedup 1.0000x reference)
import functools

import jax
import jax.numpy as jnp
from jax import lax
from jax.experimental import pallas as pl
from jax.experimental.pallas import tpu as pltpu
from jax.experimental.pallas import tpu_sc as plsc

FEAT = 32
BATCH = 16384
HIST = 50
B = BATCH * HIST
NC = 2
NS = 16
NW = NC * NS
PER_W = B // NW
GROUP = 128
NGROUPS = PER_W // GROUP
WORDS = FEAT // 2

_mesh = plsc.VectorSubcoreMesh(core_axis_name="c", subcore_axis_name="s")


@functools.partial(
    pl.kernel,
    mesh=_mesh,
    out_type=(
        jax.ShapeDtypeStruct((B, WORDS), jnp.int32),
        jax.ShapeDtypeStruct((B, WORDS), jnp.int32),
    ),
    scratch_types=[
        pltpu.VMEM((PER_W,), jnp.int32),
        pltpu.VMEM((GROUP, FEAT), jnp.float32),
        pltpu.VMEM((GROUP, FEAT), jnp.float32),
        pltpu.VMEM((GROUP, WORDS), jnp.int32),
        pltpu.VMEM((GROUP, WORDS), jnp.int32),
        pltpu.SemaphoreType.DMA,
    ],
    compiler_params=pltpu.CompilerParams(
        needs_layout_passes=False, use_tc_tiling_on_sc=False),
)
def _emb_lookup(x_hbm, rt_hbm, it_hbm, out_r_hbm, out_i_hbm,
                idx_v, rows_r, rows_i, pk_r, pk_i, sem):
    wid = lax.axis_index("s") * NC + lax.axis_index("c")
    base = wid * PER_W
    pltpu.sync_copy(x_hbm.at[pl.ds(base, PER_W)], idx_v)

    col_e = lax.iota(jnp.int32, 16) * 2
    col_o = col_e + 1
    half = jnp.full((16,), 0x8000, jnp.int32)
    himask = jnp.full((16,), -0x10000, jnp.int32)

    def group(g, carry):
        gb = g * GROUP
        pltpu.async_copy(rt_hbm.at[idx_v.at[pl.ds(gb, GROUP)]], rows_r, sem).wait()
        pltpu.async_copy(it_hbm.at[idx_v.at[pl.ds(gb, GROUP)]], rows_i, sem).wait()

        def row(r, c2):
            ridx = jnp.full((16,), r, jnp.int32)
            for rows, pk in ((rows_r, pk_r), (rows_i, pk_i)):
                lo = plsc.bitcast(plsc.load_gather(rows, [ridx, col_e]), jnp.int32)
                hi = plsc.bitcast(plsc.load_gather(rows, [ridx, col_o]), jnp.int32)
                lo_b = lax.shift_right_logical(lo + half, 16)
                hi_b = (hi + half) & himask
                pk[r] = lo_b | hi_b
            return c2
        lax.fori_loop(0, GROUP, row, 0, unroll=2)

        pltpu.sync_copy(pk_r, out_r_hbm.at[pl.ds(base + gb, GROUP)])
        pltpu.sync_copy(pk_i, out_i_hbm.at[pl.ds(base + gb, GROUP)])
        return carry

    lax.fori_loop(0, NGROUPS, group, 0)


def kernel(x, real_table, imag_table):
    pk_r, pk_i = _emb_lookup(x.reshape(-1), real_table, imag_table)

    def unpack(p):
        return lax.bitcast_convert_type(p, jnp.bfloat16).reshape(BATCH, HIST, FEAT)

    return unpack(pk_r), unpack(pk_i)
